# baseline (speedup 1.0000x reference)
import jax
import jax.numpy as jnp
from jax import lax
from jax.experimental import pallas as pl
from jax.experimental.pallas import tpu as pltpu
from jax.experimental.pallas import tpu_sc as plsc

_VOCAB = 100000
_HIDDEN = 128
_BATCH = 1024
_SEQ = 50

_NC = 2
_NS = 16
_NW = _NC * _NS
_BPW = _BATCH // _NW
_LANES = 16
_NCH = _HIDDEN // _LANES


def _pool_body(x_hbm, table_hbm, vec_hbm, idx_v, rows_v, acc_v, sem):
    wid = lax.axis_index("s") * _NC + lax.axis_index("c")
    base = wid * _BPW
    pltpu.sync_copy(x_hbm.at[pl.ds(base, _BPW)], idx_v)

    def per_row(b, carry):
        pltpu.async_copy(table_hbm.at[idx_v.at[b]], rows_v, sem).wait()

        def inner(r, accs):
            return tuple(
                accs[c] + rows_v[r, pl.ds(_LANES * c, _LANES)]
                for c in range(_NCH)
            )

        accs = lax.fori_loop(
            0, _SEQ, inner,
            tuple(jnp.zeros((_LANES,), jnp.float32) for _ in range(_NCH)),
        )
        scale = jnp.float32(1.0 / _SEQ)
        for c in range(_NCH):
            acc_v[b, pl.ds(_LANES * c, _LANES)] = accs[c] * scale
        return carry

    lax.fori_loop(0, _BPW, per_row, 0)
    pltpu.sync_copy(acc_v, vec_hbm.at[pl.ds(base, _BPW)])


def _sc_pool(x, table):
    pool = pl.kernel(
        _pool_body,
        out_type=jax.ShapeDtypeStruct((_BATCH, _HIDDEN), jnp.float32),
        mesh=plsc.VectorSubcoreMesh(core_axis_name="c", subcore_axis_name="s"),
        scratch_types=[
            pltpu.VMEM((_BPW, _SEQ), jnp.int32),
            pltpu.VMEM((_SEQ, _HIDDEN), jnp.float32),
            pltpu.VMEM((_BPW, _HIDDEN), jnp.float32),
            pltpu.SemaphoreType.DMA,
        ],
    )
    return pool(x, table)

_VT = 512


def _mm_body(vec_ref, w_ref, bias_ref, o_ref):
    o_ref[...] = lax.dot_general(
        vec_ref[...], w_ref[...],
        (((1,), (1,)), ((), ())),
        preferred_element_type=jnp.float32,
    ) + bias_ref[...]


def _matmul(vec, W, bias2d):
    return pl.pallas_call(
        _mm_body,
        grid=(pl.cdiv(_VOCAB, _VT),),
        in_specs=[
            pl.BlockSpec((_BATCH, _HIDDEN), lambda j: (0, 0)),
            pl.BlockSpec((_VT, _HIDDEN), lambda j: (j, 0)),
            pl.BlockSpec((1, _VT), lambda j: (0, j)),
        ],
        out_specs=pl.BlockSpec((_BATCH, _VT), lambda j: (0, j)),
        out_shape=jax.ShapeDtypeStruct((_BATCH, _VOCAB), jnp.float32),
    )(vec, W, bias2d)


def kernel(x, table, W, b):
    vec = _sc_pool(x, table)
    return _matmul(vec, W, b.reshape(1, _VOCAB))


# speedup vs baseline: 1.1463x; 1.1463x over previous
import jax
import jax.numpy as jnp
from jax import lax
from jax.experimental import pallas as pl
from jax.experimental.pallas import tpu as pltpu
from jax.experimental.pallas import tpu_sc as plsc

_VOCAB = 100000
_HIDDEN = 128
_BATCH = 1024
_SEQ = 50

_NC = 2
_NS = 16
_NW = _NC * _NS
_BPW = _BATCH // _NW
_LANES = 16
_NCH = _HIDDEN // _LANES


def _pool_body(x_hbm, table_hbm, vec_hbm, idx_v, rows_v, acc_v, sem):
    wid = lax.axis_index("s") * _NC + lax.axis_index("c")
    base = wid * _BPW
    pltpu.sync_copy(x_hbm.at[pl.ds(base, _BPW)], idx_v)

    def per_row(b, carry):
        pltpu.async_copy(table_hbm.at[idx_v.at[b]], rows_v, sem).wait()

        def inner(r, accs):
            return tuple(
                accs[c] + rows_v[r, pl.ds(_LANES * c, _LANES)]
                for c in range(_NCH)
            )

        accs = lax.fori_loop(
            0, _SEQ, inner,
            tuple(jnp.zeros((_LANES,), jnp.float32) for _ in range(_NCH)),
        )
        scale = jnp.float32(1.0 / _SEQ)
        for c in range(_NCH):
            acc_v[b, pl.ds(_LANES * c, _LANES)] = accs[c] * scale
        return carry

    lax.fori_loop(0, _BPW, per_row, 0)
    pltpu.sync_copy(acc_v, vec_hbm.at[pl.ds(base, _BPW)])


def _sc_pool(x, table):
    pool = pl.kernel(
        _pool_body,
        out_type=jax.ShapeDtypeStruct((_BATCH, _HIDDEN), jnp.float32),
        mesh=plsc.VectorSubcoreMesh(core_axis_name="c", subcore_axis_name="s"),
        scratch_types=[
            pltpu.VMEM((_BPW, _SEQ), jnp.int32),
            pltpu.VMEM((_SEQ, _HIDDEN), jnp.float32),
            pltpu.VMEM((_BPW, _HIDDEN), jnp.float32),
            pltpu.SemaphoreType.DMA,
        ],
    )
    return pool(x, table)

_VT = 2048


def _mm_body(vec_ref, w_ref, bias_ref, o_ref):
    o_ref[...] = lax.dot_general(
        vec_ref[...], w_ref[...],
        (((1,), (1,)), ((), ())),
        preferred_element_type=jnp.float32,
    ) + bias_ref[...]


def _matmul(vec, W, bias2d):
    return pl.pallas_call(
        _mm_body,
        grid=(pl.cdiv(_VOCAB, _VT),),
        in_specs=[
            pl.BlockSpec((_BATCH, _HIDDEN), lambda j: (0, 0)),
            pl.BlockSpec((_VT, _HIDDEN), lambda j: (j, 0)),
            pl.BlockSpec((1, _VT), lambda j: (0, j)),
        ],
        out_specs=pl.BlockSpec((_BATCH, _VT), lambda j: (0, j)),
        out_shape=jax.ShapeDtypeStruct((_BATCH, _VOCAB), jnp.float32),
    )(vec, W, bias2d)


def kernel(x, table, W, b):
    vec = _sc_pool(x, table)
    return _matmul(vec, W, b.reshape(1, _VOCAB))


# speedup vs baseline: 1.1494x; 1.0028x over previous
import jax
import jax.numpy as jnp
from jax import lax
from jax.experimental import pallas as pl
from jax.experimental.pallas import tpu as pltpu
from jax.experimental.pallas import tpu_sc as plsc

_VOCAB = 100000
_HIDDEN = 128
_BATCH = 1024
_SEQ = 50

_NC = 2
_NS = 16
_NW = _NC * _NS
_BPW = _BATCH // _NW
_LANES = 16
_NCH = _HIDDEN // _LANES


def _pool_body(x_hbm, table_hbm, vec_hbm, idx_v, rows_v, acc_v, sem):
    wid = lax.axis_index("s") * _NC + lax.axis_index("c")
    base = wid * _BPW
    pltpu.sync_copy(x_hbm.at[pl.ds(base, _BPW)], idx_v)

    def per_row(b, carry):
        pltpu.async_copy(table_hbm.at[idx_v.at[b]], rows_v, sem).wait()

        def inner(r, accs):
            return tuple(
                accs[c] + rows_v[r, pl.ds(_LANES * c, _LANES)]
                for c in range(_NCH)
            )

        accs = lax.fori_loop(
            0, _SEQ, inner,
            tuple(jnp.zeros((_LANES,), jnp.float32) for _ in range(_NCH)),
        )
        scale = jnp.float32(1.0 / _SEQ)
        for c in range(_NCH):
            acc_v[b, pl.ds(_LANES * c, _LANES)] = accs[c] * scale
        return carry

    lax.fori_loop(0, _BPW, per_row, 0)
    pltpu.sync_copy(acc_v, vec_hbm.at[pl.ds(base, _BPW)])


def _sc_pool(x, table):
    pool = pl.kernel(
        _pool_body,
        out_type=jax.ShapeDtypeStruct((_BATCH, _HIDDEN), jnp.float32),
        mesh=plsc.VectorSubcoreMesh(core_axis_name="c", subcore_axis_name="s"),
        scratch_types=[
            pltpu.VMEM((_BPW, _SEQ), jnp.int32),
            pltpu.VMEM((_SEQ, _HIDDEN), jnp.float32),
            pltpu.VMEM((_BPW, _HIDDEN), jnp.float32),
            pltpu.SemaphoreType.DMA,
        ],
    )
    return pool(x, table)

_VT = 2048


def _mm_body(vec_ref, w_ref, bias_ref, o_ref):
    o_ref[...] = lax.dot_general(
        vec_ref[...].astype(jnp.bfloat16), w_ref[...].astype(jnp.bfloat16),
        (((1,), (1,)), ((), ())),
        preferred_element_type=jnp.float32,
    ) + bias_ref[...]


def _matmul(vec, W, bias2d):
    return pl.pallas_call(
        _mm_body,
        grid=(pl.cdiv(_VOCAB, _VT),),
        in_specs=[
            pl.BlockSpec((_BATCH, _HIDDEN), lambda j: (0, 0)),
            pl.BlockSpec((_VT, _HIDDEN), lambda j: (j, 0)),
            pl.BlockSpec((1, _VT), lambda j: (0, j)),
        ],
        out_specs=pl.BlockSpec((_BATCH, _VT), lambda j: (0, j)),
        out_shape=jax.ShapeDtypeStruct((_BATCH, _VOCAB), jnp.float32),
    )(vec, W, bias2d)


def kernel(x, table, W, b):
    vec = _sc_pool(x, table)
    return _matmul(vec, W, b.reshape(1, _VOCAB))


# speedup vs baseline: 3.1267x; 2.7202x over previous
import jax
import jax.numpy as jnp
from jax import lax
from jax.experimental import pallas as pl
from jax.experimental.pallas import tpu as pltpu
from jax.experimental.pallas import tpu_sc as plsc

_VOCAB = 100000
_HIDDEN = 128
_BATCH = 1024
_SEQ = 50

_NC = 2
_NS = 16
_NW = _NC * _NS
_BPW = _BATCH // _NW
_LANES = 16
_NCH = _HIDDEN // _LANES


def _pool_body(x_hbm, table_hbm, vec_hbm, idx_v, rows_v, acc_v, sem):
    wid = lax.axis_index("s") * _NC + lax.axis_index("c")
    base = wid * _BPW
    pltpu.sync_copy(x_hbm.at[pl.ds(base, _BPW)], idx_v)

    def per_row(b, carry):
        pltpu.async_copy(table_hbm.at[idx_v.at[b]], rows_v, sem).wait()

        def inner(r, accs):
            return tuple(
                accs[c] + rows_v[r, pl.ds(_LANES * c, _LANES)]
                for c in range(_NCH)
            )

        accs = lax.fori_loop(
            0, _SEQ, inner,
            tuple(jnp.zeros((_LANES,), jnp.float32) for _ in range(_NCH)),
        )
        scale = jnp.float32(1.0 / _SEQ)
        for c in range(_NCH):
            acc_v[b, pl.ds(_LANES * c, _LANES)] = accs[c] * scale
        return carry

    lax.fori_loop(0, _BPW, per_row, 0)
    pltpu.sync_copy(acc_v, vec_hbm.at[pl.ds(base, _BPW)])


def _sc_pool(x, table):
    pool = pl.kernel(
        _pool_body,
        out_type=jax.ShapeDtypeStruct((_BATCH, _HIDDEN), jnp.float32),
        mesh=plsc.VectorSubcoreMesh(core_axis_name="c", subcore_axis_name="s"),
        scratch_types=[
            pltpu.VMEM((_BPW, _SEQ), jnp.int32),
            pltpu.VMEM((_SEQ, _HIDDEN), jnp.float32),
            pltpu.VMEM((_BPW, _HIDDEN), jnp.float32),
            pltpu.SemaphoreType.DMA,
        ],
    )
    return pool(x, table)

_VT = 2048


def _mm_body(vecT_ref, w_ref, bias_ref, o_ref):
    acc = lax.dot_general(
        w_ref[...].astype(jnp.bfloat16), vecT_ref[...].astype(jnp.bfloat16),
        (((1,), (0,)), ((), ())),
        preferred_element_type=jnp.float32,
    )
    ones = jnp.ones((1, _BATCH), jnp.float32)
    acc += lax.dot_general(
        bias_ref[...], ones, (((0,), (0,)), ((), ())),
        preferred_element_type=jnp.float32,
    )
    o_ref[...] = acc


def _matmul_t(vecT, W, bias2d):
    return pl.pallas_call(
        _mm_body,
        grid=(pl.cdiv(_VOCAB, _VT),),
        in_specs=[
            pl.BlockSpec((_HIDDEN, _BATCH), lambda j: (0, 0)),
            pl.BlockSpec((_VT, _HIDDEN), lambda j: (j, 0)),
            pl.BlockSpec((1, _VT), lambda j: (0, j)),
        ],
        out_specs=pl.BlockSpec((_VT, _BATCH), lambda j: (j, 0)),
        out_shape=jax.ShapeDtypeStruct((_VOCAB, _BATCH), jnp.float32),
    )(vecT, W, bias2d)


def kernel(x, table, W, b):
    vec = _sc_pool(x, table)
    predT = _matmul_t(vec.T, W, b.reshape(1, _VOCAB))
    return predT.T


# speedup vs baseline: 3.5283x; 1.1284x over previous
import jax
import jax.numpy as jnp
from jax import lax
from jax.experimental import pallas as pl
from jax.experimental.pallas import tpu as pltpu
from jax.experimental.pallas import tpu_sc as plsc

_VOCAB = 100000
_HIDDEN = 128
_BATCH = 1024
_SEQ = 50

_NC = 2
_NS = 16
_NW = _NC * _NS
_BPW = _BATCH // _NW
_LANES = 16
_NCH = _HIDDEN // _LANES


_NBUF = 4


def _pool_body(x_hbm, table_hbm, vec_hbm, idx_v, rows_v, acc_v, sems):
    wid = lax.axis_index("s") * _NC + lax.axis_index("c")
    base = wid * _BPW
    pltpu.sync_copy(x_hbm.at[pl.ds(base, _BPW)], idx_v)

    for k in range(_NBUF):
        pltpu.async_copy(table_hbm.at[idx_v.at[k]], rows_v.at[k], sems.at[k])

    def accumulate(b, k):
        def inner(r, accs):
            return tuple(
                accs[c] + rows_v[k, r, pl.ds(_LANES * c, _LANES)]
                for c in range(_NCH)
            )

        accs = lax.fori_loop(
            0, _SEQ, inner,
            tuple(jnp.zeros((_LANES,), jnp.float32) for _ in range(_NCH)),
            unroll=5,
        )
        scale = jnp.float32(1.0 / _SEQ)
        for c in range(_NCH):
            acc_v[b, pl.ds(_LANES * c, _LANES)] = accs[c] * scale

    def per_group(g, carry):
        for k in range(_NBUF):
            b = g * _NBUF + k
            pltpu.make_async_copy(
                table_hbm.at[idx_v.at[k]], rows_v.at[k], sems.at[k]
            ).wait()
            accumulate(b, k)

            @pl.when(b + _NBUF < _BPW)
            def _fire():
                pltpu.async_copy(
                    table_hbm.at[idx_v.at[b + _NBUF]], rows_v.at[k], sems.at[k]
                )

        return carry

    lax.fori_loop(0, _BPW // _NBUF, per_group, 0)
    pltpu.sync_copy(acc_v, vec_hbm.at[pl.ds(base, _BPW)])


def _sc_pool(x, table):
    pool = pl.kernel(
        _pool_body,
        out_type=jax.ShapeDtypeStruct((_BATCH, _HIDDEN), jnp.float32),
        mesh=plsc.VectorSubcoreMesh(core_axis_name="c", subcore_axis_name="s"),
        scratch_types=[
            pltpu.VMEM((_BPW, _SEQ), jnp.int32),
            pltpu.VMEM((_NBUF, _SEQ, _HIDDEN), jnp.float32),
            pltpu.VMEM((_BPW, _HIDDEN), jnp.float32),
            pltpu.SemaphoreType.DMA((_NBUF,)),
        ],
    )
    return pool(x, table)

_VT = 2048


def _mm_body(vecT_ref, w_ref, bias_ref, o_ref):
    acc = lax.dot_general(
        w_ref[...].astype(jnp.bfloat16), vecT_ref[...].astype(jnp.bfloat16),
        (((1,), (0,)), ((), ())),
        preferred_element_type=jnp.float32,
    )
    ones = jnp.ones((1, _BATCH), jnp.float32)
    acc += lax.dot_general(
        bias_ref[...], ones, (((0,), (0,)), ((), ())),
        preferred_element_type=jnp.float32,
    )
    o_ref[...] = acc


def _matmul_t(vecT, W, bias2d):
    return pl.pallas_call(
        _mm_body,
        grid=(pl.cdiv(_VOCAB, _VT),),
        in_specs=[
            pl.BlockSpec((_HIDDEN, _BATCH), lambda j: (0, 0)),
            pl.BlockSpec((_VT, _HIDDEN), lambda j: (j, 0)),
            pl.BlockSpec((1, _VT), lambda j: (0, j)),
        ],
        out_specs=pl.BlockSpec((_VT, _BATCH), lambda j: (j, 0)),
        out_shape=jax.ShapeDtypeStruct((_VOCAB, _BATCH), jnp.float32),
    )(vecT, W, bias2d)


def kernel(x, table, W, b):
    vec = _sc_pool(x, table)
    predT = _matmul_t(vec.T, W, b.reshape(1, _VOCAB))
    return predT.T


# speedup vs baseline: 3.5928x; 1.0183x over previous
import jax
import jax.numpy as jnp
from jax import lax
from jax.experimental import pallas as pl
from jax.experimental.pallas import tpu as pltpu
from jax.experimental.pallas import tpu_sc as plsc

_VOCAB = 100000
_HIDDEN = 128
_BATCH = 1024
_SEQ = 50

_NC = 2
_NS = 16
_NW = _NC * _NS
_BPW = _BATCH // _NW
_LANES = 16
_NCH = _HIDDEN // _LANES


_NBUF = 4


def _pool_body(x_hbm, table_hbm, vec_hbm, idx_v, rows_v, acc_v, sems):
    wid = lax.axis_index("s") * _NC + lax.axis_index("c")
    base = wid * _BPW
    pltpu.sync_copy(x_hbm.at[pl.ds(base, _BPW)], idx_v)

    for k in range(_NBUF):
        pltpu.async_copy(table_hbm.at[idx_v.at[k]], rows_v.at[k], sems.at[k])

    def accumulate(b, k):
        def inner(r, accs):
            return tuple(
                accs[c] + rows_v[k, r, pl.ds(_LANES * c, _LANES)]
                for c in range(_NCH)
            )

        accs = lax.fori_loop(
            0, _SEQ, inner,
            tuple(jnp.zeros((_LANES,), jnp.float32) for _ in range(_NCH)),
            unroll=5,
        )
        scale = jnp.float32(1.0 / _SEQ)
        for c in range(_NCH):
            acc_v[b, pl.ds(_LANES * c, _LANES)] = accs[c] * scale

    def per_group(g, carry):
        for k in range(_NBUF):
            b = g * _NBUF + k
            pltpu.make_async_copy(
                table_hbm.at[idx_v.at[k]], rows_v.at[k], sems.at[k]
            ).wait()
            accumulate(b, k)

            @pl.when(b + _NBUF < _BPW)
            def _fire():
                pltpu.async_copy(
                    table_hbm.at[idx_v.at[b + _NBUF]], rows_v.at[k], sems.at[k]
                )

        return carry

    lax.fori_loop(0, _BPW // _NBUF, per_group, 0)
    pltpu.sync_copy(acc_v, vec_hbm.at[pl.ds(base, _BPW)])


def _sc_pool(x, table):
    pool = pl.kernel(
        _pool_body,
        out_type=jax.ShapeDtypeStruct((_BATCH, _HIDDEN), jnp.float32),
        mesh=plsc.VectorSubcoreMesh(core_axis_name="c", subcore_axis_name="s"),
        scratch_types=[
            pltpu.VMEM((_BPW, _SEQ), jnp.int32),
            pltpu.VMEM((_NBUF, _SEQ, _HIDDEN), jnp.float32),
            pltpu.VMEM((_BPW, _HIDDEN), jnp.float32),
            pltpu.SemaphoreType.DMA((_NBUF,)),
        ],
    )
    return pool(x, table)

_VT = 4096


def _mm_body(vecT_ref, w_ref, bias_ref, o_ref):
    acc = lax.dot_general(
        w_ref[...].astype(jnp.bfloat16), vecT_ref[...].astype(jnp.bfloat16),
        (((1,), (0,)), ((), ())),
        preferred_element_type=jnp.float32,
    )
    ones = jnp.ones((1, _BATCH), jnp.float32)
    acc += lax.dot_general(
        bias_ref[...], ones, (((0,), (0,)), ((), ())),
        preferred_element_type=jnp.float32,
    )
    o_ref[...] = acc


def _matmul_t(vecT, W, bias2d):
    return pl.pallas_call(
        _mm_body,
        grid=(pl.cdiv(_VOCAB, _VT),),
        in_specs=[
            pl.BlockSpec((_HIDDEN, _BATCH), lambda j: (0, 0)),
            pl.BlockSpec((_VT, _HIDDEN), lambda j: (j, 0)),
            pl.BlockSpec((1, _VT), lambda j: (0, j)),
        ],
        out_specs=pl.BlockSpec((_VT, _BATCH), lambda j: (j, 0)),
        out_shape=jax.ShapeDtypeStruct((_VOCAB, _BATCH), jnp.float32),
    )(vecT, W, bias2d)


def kernel(x, table, W, b):
    vec = _sc_pool(x, table)
    predT = _matmul_t(vec.T, W, b.reshape(1, _VOCAB))
    return predT.T


# speedup vs baseline: 3.6064x; 1.0038x over previous
import jax
import jax.numpy as jnp
from jax import lax
from jax.experimental import pallas as pl
from jax.experimental.pallas import tpu as pltpu
from jax.experimental.pallas import tpu_sc as plsc

_VOCAB = 100000
_HIDDEN = 128
_BATCH = 1024
_SEQ = 50

_NC = 2
_NS = 16
_NW = _NC * _NS
_BPW = _BATCH // _NW
_LANES = 16
_NCH = _HIDDEN // _LANES


_NBUF = 4


def _pool_body(x_hbm, table_hbm, vec_hbm, idx_v, rows_v, acc_v, sems):
    wid = lax.axis_index("s") * _NC + lax.axis_index("c")
    base = wid * _BPW
    pltpu.sync_copy(x_hbm.at[pl.ds(base, _BPW)], idx_v)

    for k in range(_NBUF):
        pltpu.async_copy(table_hbm.at[idx_v.at[k]], rows_v.at[k], sems.at[k])

    def accumulate(b, k):
        def inner(r, accs):
            return tuple(
                accs[c] + rows_v[k, r, pl.ds(_LANES * c, _LANES)]
                for c in range(_NCH)
            )

        accs = lax.fori_loop(
            0, _SEQ, inner,
            tuple(jnp.zeros((_LANES,), jnp.float32) for _ in range(_NCH)),
            unroll=5,
        )
        scale = jnp.float32(1.0 / _SEQ)
        for c in range(_NCH):
            acc_v[b, pl.ds(_LANES * c, _LANES)] = accs[c] * scale

    def per_group(g, carry):
        for k in range(_NBUF):
            b = g * _NBUF + k
            pltpu.make_async_copy(
                table_hbm.at[idx_v.at[k]], rows_v.at[k], sems.at[k]
            ).wait()
            accumulate(b, k)

            @pl.when(b + _NBUF < _BPW)
            def _fire():
                pltpu.async_copy(
                    table_hbm.at[idx_v.at[b + _NBUF]], rows_v.at[k], sems.at[k]
                )

        return carry

    lax.fori_loop(0, _BPW // _NBUF, per_group, 0)
    pltpu.sync_copy(acc_v, vec_hbm.at[pl.ds(base, _BPW)])


def _sc_pool(x, table):
    pool = pl.kernel(
        _pool_body,
        out_type=jax.ShapeDtypeStruct((_BATCH, _HIDDEN), jnp.float32),
        mesh=plsc.VectorSubcoreMesh(core_axis_name="c", subcore_axis_name="s"),
        scratch_types=[
            pltpu.VMEM((_BPW, _SEQ), jnp.int32),
            pltpu.VMEM((_NBUF, _SEQ, _HIDDEN), jnp.float32),
            pltpu.VMEM((_BPW, _HIDDEN), jnp.float32),
            pltpu.SemaphoreType.DMA((_NBUF,)),
        ],
    )
    return pool(x, table)

_VT = 5120


def _mm_body(vecT_ref, w_ref, bias_ref, o_ref):
    acc = lax.dot_general(
        w_ref[...].astype(jnp.bfloat16), vecT_ref[...].astype(jnp.bfloat16),
        (((1,), (0,)), ((), ())),
        preferred_element_type=jnp.float32,
    )
    ones = jnp.ones((1, _BATCH), jnp.float32)
    acc += lax.dot_general(
        bias_ref[...], ones, (((0,), (0,)), ((), ())),
        preferred_element_type=jnp.float32,
    )
    o_ref[...] = acc


def _matmul_t(vecT, W, bias2d):
    return pl.pallas_call(
        _mm_body,
        grid=(pl.cdiv(_VOCAB, _VT),),
        in_specs=[
            pl.BlockSpec((_HIDDEN, _BATCH), lambda j: (0, 0)),
            pl.BlockSpec((_VT, _HIDDEN), lambda j: (j, 0)),
            pl.BlockSpec((1, _VT), lambda j: (0, j)),
        ],
        out_specs=pl.BlockSpec((_VT, _BATCH), lambda j: (j, 0)),
        out_shape=jax.ShapeDtypeStruct((_VOCAB, _BATCH), jnp.float32),
    )(vecT, W, bias2d)


def kernel(x, table, W, b):
    vec = _sc_pool(x, table)
    predT = _matmul_t(vec.T, W, b.reshape(1, _VOCAB))
    return predT.T
